# baseline (speedup 1.0000x reference)
import functools

import jax
import jax.numpy as jnp
import numpy as np
from jax import lax
from jax.experimental import pallas as pl
from jax.experimental.pallas import tpu as pltpu
from jax.experimental.pallas import tpu_sc as plsc

NUM_RESAMPLE = 64
TINY = np.float32(1.1754944e-38)
K2 = 42
K3 = 0 ^ K2 ^ 0x1BD11BDA


def _rotl(x, d):
    return lax.shift_left(x, jnp.uint32(d)) | lax.shift_right_logical(
        x, jnp.uint32(32 - d))


def _threefry_bits(x1):
    ks = (jnp.uint32(0), jnp.uint32(K2), jnp.uint32(K3))
    rot = ((13, 15, 26, 6), (17, 29, 16, 24))
    x0 = x1
    x1 = _rotl(x1, 13) ^ x0
    for r in rot[0][1:]:
        x0 = x0 + x1
        x1 = _rotl(x1, r)
        x1 = x1 ^ x0
    x0 = x0 + ks[1]
    x1 = x1 + jnp.uint32(K3 + 1)
    for d in range(1, 5):
        for r in rot[d % 2]:
            x0 = x0 + x1
            x1 = _rotl(x1, r)
            x1 = x1 ^ x0
        x0 = x0 + ks[(d + 1) % 3]
        x1 = x1 + ks[(d + 2) % 3] + jnp.uint32(d + 1)
    return x0 ^ x1


def _sample_body(w_ref, inds_ref, norm_ref, *, br, s_dim):
    blk = pl.program_id(0)
    j_iota = lax.broadcasted_iota(jnp.uint32, (NUM_RESAMPLE, s_dim), 0)
    s_iota = lax.broadcasted_iota(jnp.uint32, (NUM_RESAMPLE, s_dim), 1)
    off = s_iota * jnp.uint32(NUM_RESAMPLE) + j_iota
    lane_i = lax.broadcasted_iota(jnp.int32, (NUM_RESAMPLE, s_dim), 1)
    per_ray = jnp.uint32(s_dim * NUM_RESAMPLE)
    for rloc in range(br):
        r = (blk * br + rloc).astype(jnp.uint32)
        w = w_ref[rloc, :].reshape(1, s_dim)
        wm = jnp.maximum(w, TINY)
        neginvw = jnp.float32(-1.0) / wm
        norm = jnp.sum(wm, keepdims=True)
        x1 = (r * per_ray + jnp.uint32(K2)) + off
        bits = _threefry_bits(x1)
        fb = lax.shift_right_logical(bits, jnp.uint32(9)) | jnp.uint32(
            0x3F800000)
        u = lax.bitcast_convert_type(fb, jnp.float32) - jnp.float32(1.0)
        val = jnp.log(u) * neginvw
        m = jnp.min(val, axis=1, keepdims=True)
        idx = jnp.min(jnp.where(val == m, lane_i, s_dim), axis=1,
                      keepdims=True)
        gidx = idx + (blk * br + rloc) * s_dim
        inds_ref[0, :, rloc:rloc + 1] = gidx
        norm_ref[0, 0:1, rloc:rloc + 1] = norm


@functools.partial(jax.jit, static_argnames=("interpret",))
def _sample(weights, interpret=False):
    n_rays, s_dim = weights.shape
    br = 16
    body = functools.partial(_sample_body, br=br, s_dim=s_dim)
    inds_t, norm_t = pl.pallas_call(
        body,
        grid=(n_rays // br,),
        in_specs=[pl.BlockSpec((br, s_dim), lambda i: (i, 0))],
        out_specs=[
            pl.BlockSpec((1, NUM_RESAMPLE, br), lambda i: (i, 0, 0)),
            pl.BlockSpec((1, 1, br), lambda i: (i, 0, 0)),
        ],
        out_shape=[
            jax.ShapeDtypeStruct((n_rays // br, NUM_RESAMPLE, br), jnp.int32),
            jax.ShapeDtypeStruct((n_rays // br, 1, br), jnp.float32),
        ],
        interpret=interpret,
    )(weights)
    inds = inds_t.transpose(0, 2, 1).reshape(n_rays * NUM_RESAMPLE)
    norm = norm_t.reshape(n_rays // br, br).reshape(n_rays, 1)
    return inds, norm


def _gather_sc(pairs, idx2d):
    n_tot = idx2d.shape[0]
    ch = 128
    info = plsc.get_sparse_core_info()
    n_workers = info.num_cores * info.num_subcores
    n_ch = n_tot // n_workers
    mesh = plsc.VectorSubcoreMesh(core_axis_name="c", subcore_axis_name="s")

    @functools.partial(
        pl.kernel,
        mesh=mesh,
        compiler_params=pltpu.CompilerParams(use_tc_tiling_on_sc=False),
        out_type=jax.ShapeDtypeStruct((n_tot, ch, 8), jnp.float32),
        scratch_types=[
            pltpu.VMEM((ch,), jnp.int32),
            pltpu.VMEM((ch, 8), jnp.float32),
            pltpu.SemaphoreType.DMA,
        ],
    )
    def k(p_hbm, idx_hbm, out_hbm, idx_v, rows_v, sem):
        wid = lax.axis_index("s") * info.num_cores + lax.axis_index("c")

        def body(j, carry):
            row = wid * n_ch + j
            pltpu.sync_copy(idx_hbm.at[row], idx_v)
            pltpu.async_copy(p_hbm.at[idx_v], rows_v, sem).wait()
            pltpu.sync_copy(rows_v, out_hbm.at[row])
            return carry

        lax.fori_loop(0, n_ch, body, 0)

    return k(pairs, idx2d)


def _reweight_body(g_ref, n_ref, out_ref):
    v = g_ref[...]
    s = n_ref[...]
    lane = lax.broadcasted_iota(jnp.int32, v.shape, 1)
    p = jnp.maximum(v, TINY) / s
    neww = v / (jnp.float32(NUM_RESAMPLE) * p + jnp.float32(1e-8))
    out_ref[...] = jnp.where((lane & 3) == 3, neww, v)


@functools.partial(jax.jit, static_argnames=("interpret",))
def _reweight(g4, norm, interpret=False):
    n_rays = norm.shape[0]
    bn = min(256, n_rays)
    flat = g4.reshape(n_rays, NUM_RESAMPLE * 4)
    out = pl.pallas_call(
        _reweight_body,
        grid=(n_rays // bn,),
        in_specs=[
            pl.BlockSpec((bn, NUM_RESAMPLE * 4), lambda i: (i, 0)),
            pl.BlockSpec((bn, 1), lambda i: (i, 0)),
        ],
        out_specs=pl.BlockSpec((bn, NUM_RESAMPLE * 4), lambda i: (i, 0)),
        out_shape=jax.ShapeDtypeStruct((n_rays, NUM_RESAMPLE * 4),
                                       jnp.float32),
        interpret=interpret,
    )(flat, norm)
    return out.reshape(n_rays, NUM_RESAMPLE, 4)


def kernel(weights, points):
    n_rays, s_dim = weights.shape
    n_out = n_rays * NUM_RESAMPLE
    inds, norm = _sample(weights)
    p4 = jnp.concatenate([points, weights[..., None]], axis=-1)
    pairs = p4.reshape(n_rays * s_dim // 2, 8)
    g8 = _gather_sc(pairs, (inds >> 1).reshape(n_out // 128, 128))
    g8 = g8.reshape(n_out, 8)
    g4 = jnp.where((inds & 1)[:, None] == 1, g8[:, 4:], g8[:, :4])
    return _reweight(g4, norm)


# speedup vs baseline: 1.0047x; 1.0047x over previous
import functools

import jax
import jax.numpy as jnp
import numpy as np
from jax import lax
from jax.experimental import pallas as pl
from jax.experimental.pallas import tpu as pltpu
from jax.experimental.pallas import tpu_sc as plsc

NUM_RESAMPLE = 64
TINY = np.float32(1.1754944e-38)
K2 = 42
K3 = 0 ^ K2 ^ 0x1BD11BDA


def _rotl(x, d):
    return lax.shift_left(x, jnp.uint32(d)) | lax.shift_right_logical(
        x, jnp.uint32(32 - d))


def _threefry_bits(x1):
    ks = (jnp.uint32(0), jnp.uint32(K2), jnp.uint32(K3))
    rot = ((13, 15, 26, 6), (17, 29, 16, 24))
    x0 = x1
    x1 = _rotl(x1, 13) ^ x0
    for r in rot[0][1:]:
        x0 = x0 + x1
        x1 = _rotl(x1, r)
        x1 = x1 ^ x0
    x0 = x0 + ks[1]
    x1 = x1 + jnp.uint32(K3 + 1)
    for d in range(1, 5):
        for r in rot[d % 2]:
            x0 = x0 + x1
            x1 = _rotl(x1, r)
            x1 = x1 ^ x0
        x0 = x0 + ks[(d + 1) % 3]
        x1 = x1 + ks[(d + 2) % 3] + jnp.uint32(d + 1)
    return x0 ^ x1


def _sample_body(w_ref, inds_ref, norm_ref, *, br, s_dim):
    blk = pl.program_id(0)
    j_iota = lax.broadcasted_iota(jnp.uint32, (NUM_RESAMPLE, s_dim), 0)
    s_iota = lax.broadcasted_iota(jnp.uint32, (NUM_RESAMPLE, s_dim), 1)
    off = s_iota * jnp.uint32(NUM_RESAMPLE) + j_iota
    lane_i = lax.broadcasted_iota(jnp.int32, (NUM_RESAMPLE, s_dim), 1)
    per_ray = jnp.uint32(s_dim * NUM_RESAMPLE)
    for rloc in range(br):
        r = (blk * br + rloc).astype(jnp.uint32)
        w = w_ref[rloc, :].reshape(1, s_dim)
        wm = jnp.maximum(w, TINY)
        neginvw = jnp.float32(-1.0) / wm
        norm = jnp.sum(wm, keepdims=True)
        x1 = (r * per_ray + jnp.uint32(K2)) + off
        bits = _threefry_bits(x1)
        fb = lax.shift_right_logical(bits, jnp.uint32(9)) | jnp.uint32(
            0x3F800000)
        u = lax.bitcast_convert_type(fb, jnp.float32) - jnp.float32(1.0)
        val = jnp.log(u) * neginvw
        m = jnp.min(val, axis=1, keepdims=True)
        idx = jnp.min(jnp.where(val == m, lane_i, s_dim), axis=1,
                      keepdims=True)
        gidx = idx + (blk * br + rloc) * s_dim
        inds_ref[0, :, rloc:rloc + 1] = gidx
        norm_ref[0, 0:1, rloc:rloc + 1] = norm


@functools.partial(jax.jit, static_argnames=("interpret",))
def _sample(weights, interpret=False):
    n_rays, s_dim = weights.shape
    br = 16
    body = functools.partial(_sample_body, br=br, s_dim=s_dim)
    inds_t, norm_t = pl.pallas_call(
        body,
        grid=(n_rays // br,),
        in_specs=[pl.BlockSpec((br, s_dim), lambda i: (i, 0))],
        out_specs=[
            pl.BlockSpec((1, NUM_RESAMPLE, br), lambda i: (i, 0, 0)),
            pl.BlockSpec((1, 1, br), lambda i: (i, 0, 0)),
        ],
        out_shape=[
            jax.ShapeDtypeStruct((n_rays // br, NUM_RESAMPLE, br), jnp.int32),
            jax.ShapeDtypeStruct((n_rays // br, 1, br), jnp.float32),
        ],
        interpret=interpret,
    )(weights)
    inds = inds_t.transpose(0, 2, 1).reshape(n_rays * NUM_RESAMPLE)
    norm = norm_t.reshape(n_rays // br, br).reshape(n_rays, 1)
    return inds, norm


def _gather_sc(pairs, idx2d):
    n_tot = idx2d.shape[0]
    ch = 128
    info = plsc.get_sparse_core_info()
    n_workers = info.num_cores * info.num_subcores
    n_ch = n_tot // n_workers
    mesh = plsc.VectorSubcoreMesh(core_axis_name="c", subcore_axis_name="s")

    @functools.partial(
        pl.kernel,
        mesh=mesh,
        compiler_params=pltpu.CompilerParams(use_tc_tiling_on_sc=False),
        out_type=jax.ShapeDtypeStruct((n_tot, ch, 8), jnp.float32),
        scratch_types=[
            pltpu.VMEM((n_ch, ch), jnp.int32),
            pltpu.VMEM((n_ch, ch, 8), jnp.float32),
            pltpu.SemaphoreType.DMA,
        ],
    )
    def k(p_hbm, idx_hbm, out_hbm, idx_v, rows_v, sem):
        wid = lax.axis_index("s") * info.num_cores + lax.axis_index("c")
        base = wid * n_ch
        pltpu.sync_copy(idx_hbm.at[pl.ds(base, n_ch)], idx_v)
        wave = 8
        n_w = n_ch // wave

        def fire(w):
            for b in range(wave):
                j = w * wave + b
                pltpu.async_copy(p_hbm.at[idx_v.at[j]], rows_v.at[j], sem)

        def drain(w):
            for b in range(wave):
                j = w * wave + b
                pltpu.make_async_copy(
                    p_hbm.at[pl.ds(0, ch)], rows_v.at[j], sem).wait()

        fire(0)

        def body(w, carry):
            fire(w + 1)
            drain(w)
            return carry

        lax.fori_loop(0, n_w - 1, body, 0)
        drain(n_w - 1)
        pltpu.sync_copy(rows_v, out_hbm.at[pl.ds(base, n_ch)])

    return k(pairs, idx2d)


def _reweight_body(g_ref, n_ref, out_ref):
    v = g_ref[...]
    s = n_ref[...]
    lane = lax.broadcasted_iota(jnp.int32, v.shape, 1)
    p = jnp.maximum(v, TINY) / s
    neww = v / (jnp.float32(NUM_RESAMPLE) * p + jnp.float32(1e-8))
    out_ref[...] = jnp.where((lane & 3) == 3, neww, v)


@functools.partial(jax.jit, static_argnames=("interpret",))
def _reweight(g4, norm, interpret=False):
    n_rays = norm.shape[0]
    bn = min(256, n_rays)
    flat = g4.reshape(n_rays, NUM_RESAMPLE * 4)
    out = pl.pallas_call(
        _reweight_body,
        grid=(n_rays // bn,),
        in_specs=[
            pl.BlockSpec((bn, NUM_RESAMPLE * 4), lambda i: (i, 0)),
            pl.BlockSpec((bn, 1), lambda i: (i, 0)),
        ],
        out_specs=pl.BlockSpec((bn, NUM_RESAMPLE * 4), lambda i: (i, 0)),
        out_shape=jax.ShapeDtypeStruct((n_rays, NUM_RESAMPLE * 4),
                                       jnp.float32),
        interpret=interpret,
    )(flat, norm)
    return out.reshape(n_rays, NUM_RESAMPLE, 4)


def kernel(weights, points):
    n_rays, s_dim = weights.shape
    n_out = n_rays * NUM_RESAMPLE
    inds, norm = _sample(weights)
    p4 = jnp.concatenate([points, weights[..., None]], axis=-1)
    pairs = p4.reshape(n_rays * s_dim // 2, 8)
    g8 = _gather_sc(pairs, (inds >> 1).reshape(n_out // 128, 128))
    g8 = g8.reshape(n_out, 8)
    g4 = jnp.where((inds & 1)[:, None] == 1, g8[:, 4:], g8[:, :4])
    return _reweight(g4, norm)
